# baseline (speedup 1.0000x reference)
import functools

import jax
import jax.numpy as jnp
from jax import lax
from jax.experimental import pallas as pl
from jax.experimental.pallas import tpu as pltpu
from jax.experimental.pallas import tpu_sc as plsc

NUM_ROLES = 6
D = 128
ROWS = 16384
COLS = 200
B = ROWS * COLS
Q = 4
B4 = B // Q
NQR = NUM_ROLES ** Q

NC = 2
NS = 16
NW = NC * NS
B4_PER_W = B4 // NW

BLK = 64
N_BLK = B4_PER_W // BLK
NBUF = 2


PACK_LANES = 512
PACK_ROWS = 800
PACK_GRID = B // (PACK_ROWS * PACK_LANES)


def _pack_body(idx_ref, out_ref):
    x = idx_ref[...].astype(jnp.float32)
    l = lax.broadcasted_iota(jnp.int32, (PACK_LANES, PACK_LANES // Q), 0)
    q = lax.broadcasted_iota(jnp.int32, (PACK_LANES, PACK_LANES // Q), 1)
    p = l - q * Q
    w = ((p == 0) * 216 + (p == 1) * 36 + (p == 2) * 6 + (p == 3) * 1)
    w = jnp.where(l // Q == q, w, 0).astype(jnp.float32)
    packed = lax.dot_general(x, w, (((1,), (0,)), ((), ())),
                             precision=lax.Precision.HIGHEST)
    out_ref[...] = packed.astype(jnp.int32)


def _pack_quads(flat_idx):
    idx2d = flat_idx.reshape(B // PACK_LANES, PACK_LANES)
    out = pl.pallas_call(
        _pack_body,
        grid=(PACK_GRID,),
        in_specs=[pl.BlockSpec((PACK_ROWS, PACK_LANES), lambda i: (i, 0))],
        out_specs=pl.BlockSpec((PACK_ROWS, PACK_LANES // Q), lambda i: (i, 0)),
        out_shape=jax.ShapeDtypeStruct((B // PACK_LANES, PACK_LANES // Q),
                                       jnp.int32),
    )(idx2d)
    return out.reshape(B4)


@functools.partial(
    pl.kernel,
    mesh=plsc.VectorSubcoreMesh(core_axis_name="c", subcore_axis_name="s"),
    out_type=jax.ShapeDtypeStruct((B4, Q, D), jnp.float32),
    scratch_types=[
        pltpu.VMEM((NBUF, BLK), jnp.int32),
        pltpu.VMEM((NBUF, BLK, Q, D), jnp.float32),
        pltpu.VMEM_SHARED((NQR, Q, D), jnp.float32),
        pltpu.SemaphoreType.DMA,
        pltpu.SemaphoreType.DMA,
    ],
)
def _gather_rows(idx_hbm, table4_hbm, out_hbm, idx_v, rows_v, table_v,
                 sem_g, sem_w):
    wid = lax.axis_index("s") * NC + lax.axis_index("c")
    base = wid * B4_PER_W
    @pl.when(lax.axis_index("s") == 0)
    def _():
        pltpu.sync_copy(table4_hbm, table_v)

    plsc.subcore_barrier()

    def fire_gather(b):
        pltpu.async_copy(table_v.at[idx_v.at[b]], rows_v.at[b], sem_g)

    def drain_blk(sem, b):
        pltpu.make_async_copy(out_hbm.at[pl.ds(0, BLK)], rows_v.at[b],
                              sem).wait()

    pltpu.sync_copy(idx_hbm.at[pl.ds(base, BLK)], idx_v.at[0])
    fire_gather(0)
    pltpu.sync_copy(idx_hbm.at[pl.ds(base + BLK, BLK)], idx_v.at[1])

    def step(i, carry):
        b = lax.rem(i, NBUF)
        b1 = lax.rem(i + 1, NBUF)

        @pl.when(i >= 1)
        def _():
            drain_blk(sem_w, b1)

        @pl.when(i < N_BLK - 1)
        def _():
            fire_gather(b1)

        drain_blk(sem_g, b)
        pltpu.async_copy(rows_v.at[b], out_hbm.at[pl.ds(base + i * BLK, BLK)],
                         sem_w)

        @pl.when(i + 2 < N_BLK)
        def _():
            pltpu.sync_copy(idx_hbm.at[pl.ds(base + (i + 2) * BLK, BLK)],
                            idx_v.at[lax.rem(i + 2, NBUF)])

        return carry

    lax.fori_loop(0, N_BLK, step, 0)
    drain_blk(sem_w, (N_BLK - 1) % NBUF)


def kernel(role_indices, embedding_weight):
    flat_idx = role_indices.reshape(B).astype(jnp.int32)
    idx4 = _pack_quads(flat_idx)
    r = jnp.arange(NQR, dtype=jnp.int32)
    digits = jnp.stack([(r // 216) % 6, (r // 36) % 6, (r // 6) % 6, r % 6],
                       axis=1)
    table4 = embedding_weight[digits]
    out = _gather_rows(idx4, table4)
    return out.reshape(ROWS, COLS, D)


# speedup vs baseline: 1.0122x; 1.0122x over previous
import functools

import jax
import jax.numpy as jnp
from jax import lax
from jax.experimental import pallas as pl
from jax.experimental.pallas import tpu as pltpu
from jax.experimental.pallas import tpu_sc as plsc

NUM_ROLES = 6
D = 128
ROWS = 16384
COLS = 200
B = ROWS * COLS
Q = 4
B4 = B // Q
NQR = NUM_ROLES ** Q

NC = 2
NS = 16
NW = NC * NS
B4_PER_W = B4 // NW

BLK = 80
N_BLK = B4_PER_W // BLK
NBUF = 2


PACK_LANES = 512
PACK_ROWS = 800
PACK_GRID = B // (PACK_ROWS * PACK_LANES)


def _pack_body(idx_ref, out_ref):
    x = idx_ref[...].astype(jnp.float32)
    l = lax.broadcasted_iota(jnp.int32, (PACK_LANES, PACK_LANES // Q), 0)
    q = lax.broadcasted_iota(jnp.int32, (PACK_LANES, PACK_LANES // Q), 1)
    p = l - q * Q
    w = ((p == 0) * 216 + (p == 1) * 36 + (p == 2) * 6 + (p == 3) * 1)
    w = jnp.where(l // Q == q, w, 0).astype(jnp.float32)
    packed = lax.dot_general(x, w, (((1,), (0,)), ((), ())),
                             precision=lax.Precision.HIGHEST)
    out_ref[...] = packed.astype(jnp.int32)


def _pack_quads(flat_idx):
    idx2d = flat_idx.reshape(B // PACK_LANES, PACK_LANES)
    out = pl.pallas_call(
        _pack_body,
        grid=(PACK_GRID,),
        in_specs=[pl.BlockSpec((PACK_ROWS, PACK_LANES), lambda i: (i, 0))],
        out_specs=pl.BlockSpec((PACK_ROWS, PACK_LANES // Q), lambda i: (i, 0)),
        out_shape=jax.ShapeDtypeStruct((B // PACK_LANES, PACK_LANES // Q),
                                       jnp.int32),
    )(idx2d)
    return out.reshape(B4)


@functools.partial(
    pl.kernel,
    mesh=plsc.VectorSubcoreMesh(core_axis_name="c", subcore_axis_name="s"),
    out_type=jax.ShapeDtypeStruct((B4, Q, D), jnp.float32),
    scratch_types=[
        pltpu.VMEM((NBUF, BLK), jnp.int32),
        pltpu.VMEM((NBUF, BLK, Q, D), jnp.float32),
        pltpu.VMEM_SHARED((NQR, Q, D), jnp.float32),
        pltpu.SemaphoreType.DMA,
        pltpu.SemaphoreType.DMA,
    ],
)
def _gather_rows(idx_hbm, table4_hbm, out_hbm, idx_v, rows_v, table_v,
                 sem_g, sem_w):
    wid = lax.axis_index("s") * NC + lax.axis_index("c")
    base = wid * B4_PER_W
    @pl.when(lax.axis_index("s") == 0)
    def _():
        pltpu.sync_copy(table4_hbm, table_v)

    plsc.subcore_barrier()

    def fire_gather(b):
        pltpu.async_copy(table_v.at[idx_v.at[b]], rows_v.at[b], sem_g)

    def drain_blk(sem, b):
        pltpu.make_async_copy(out_hbm.at[pl.ds(0, BLK)], rows_v.at[b],
                              sem).wait()

    pltpu.sync_copy(idx_hbm.at[pl.ds(base, BLK)], idx_v.at[0])
    fire_gather(0)
    pltpu.sync_copy(idx_hbm.at[pl.ds(base + BLK, BLK)], idx_v.at[1])

    def step(i, carry):
        b = lax.rem(i, NBUF)
        b1 = lax.rem(i + 1, NBUF)
        drain_blk(sem_g, b)

        @pl.when(i >= 1)
        def _():
            drain_blk(sem_w, b1)

        @pl.when(i < N_BLK - 1)
        def _():
            fire_gather(b1)

        pltpu.async_copy(rows_v.at[b], out_hbm.at[pl.ds(base + i * BLK, BLK)],
                         sem_w)

        @pl.when(i + 2 < N_BLK)
        def _():
            pltpu.sync_copy(idx_hbm.at[pl.ds(base + (i + 2) * BLK, BLK)],
                            idx_v.at[lax.rem(i + 2, NBUF)])

        return carry

    lax.fori_loop(0, N_BLK, step, 0)
    drain_blk(sem_w, (N_BLK - 1) % NBUF)


def kernel(role_indices, embedding_weight):
    flat_idx = role_indices.reshape(B).astype(jnp.int32)
    idx4 = _pack_quads(flat_idx)
    r = jnp.arange(NQR, dtype=jnp.int32)
    digits = jnp.stack([(r // 216) % 6, (r // 36) % 6, (r // 6) % 6, r % 6],
                       axis=1)
    table4 = embedding_weight[digits]
    out = _gather_rows(idx4, table4)
    return out.reshape(ROWS, COLS, D)
